# baseline (speedup 1.0000x reference)
import functools

import numpy as np
import jax
import jax.numpy as jnp
from jax import lax
from jax.experimental import pallas as pl
from jax.experimental.pallas import tpu as pltpu
from jax.experimental.pallas import tpu_sc as plsc

VOCAB_SIZE = 1000000
D_DIM = 64
BATCH_N = 4096
SEQ_L = 200


def _pe_table() -> np.ndarray:
    pos = np.arange(SEQ_L)[:, np.newaxis].astype(np.float64)
    i = np.arange(D_DIM)[np.newaxis, :].astype(np.float64)
    angle_rates = 1.0 / np.power(10000, 2 * (i // 2) / np.float32(D_DIM))
    angle_rads = pos * angle_rates
    angle_rads[:, 0::2] = np.sin(angle_rads[:, 0::2])
    angle_rads[:, 1::2] = np.cos(angle_rads[:, 1::2])
    return np.asarray(angle_rads, dtype=np.float32)


_PE_CONST = _pe_table()

_INFO = plsc.get_sparse_core_info()
_NC, _NS = _INFO.num_cores, _INFO.num_subcores
NW = _NC * _NS

N_ROWS = BATCH_N * SEQ_L
PER_W = N_ROWS // NW
SEQ_PER_CHUNK = 4
CHUNK = SEQ_PER_CHUNK * SEQ_L
NCHUNK = PER_W // CHUNK
_SUBS = [(0, 128), (128, 72)]
LANES = 16
VECS_PER_ROW = D_DIM // LANES


def _sc_embed(table, idx_flat, pe):
    mesh = plsc.VectorSubcoreMesh(core_axis_name="c", subcore_axis_name="s")

    @functools.partial(
        pl.kernel,
        mesh=mesh,
        out_type=jax.ShapeDtypeStruct((N_ROWS, D_DIM), jnp.float32),
        scratch_types=[
            pltpu.VMEM((SEQ_PER_CHUNK, SEQ_L), jnp.int32),
            pltpu.VMEM((SEQ_PER_CHUNK, SEQ_L), jnp.int32),
            pltpu.VMEM((CHUNK, D_DIM), jnp.float32),
            pltpu.VMEM((CHUNK, D_DIM), jnp.float32),
            pltpu.VMEM((SEQ_L, D_DIM), jnp.float32),
            pltpu.SemaphoreType.DMA,
            pltpu.SemaphoreType.DMA,
            pltpu.SemaphoreType.DMA,
            pltpu.SemaphoreType.DMA,
        ],
        compiler_params=pltpu.CompilerParams(use_tc_tiling_on_sc=False),
    )
    def body(table_hbm, idx_hbm, pe_hbm, out_hbm,
             idx_a, idx_b, rows_a, rows_b, pe_v,
             gsem_a, gsem_b, osem_a, osem_b):
        wid = lax.axis_index("s") * _NC + lax.axis_index("c")
        base = wid * PER_W
        pltpu.sync_copy(pe_hbm, pe_v)

        def fire(g, idx_v, rows_v, gsem):
            seq0 = base // SEQ_L + g * SEQ_PER_CHUNK
            pltpu.sync_copy(idx_hbm.at[pl.ds(seq0, SEQ_PER_CHUNK)], idx_v)
            for s in range(SEQ_PER_CHUNK):
                for off, ln in _SUBS:
                    pltpu.async_copy(
                        table_hbm.at[idx_v.at[s].at[pl.ds(off, ln)]],
                        rows_v.at[pl.ds(s * SEQ_L + off, ln)],
                        gsem,
                    )

        def wait_g(idx_v, rows_v, gsem):
            for s in range(SEQ_PER_CHUNK):
                for off, ln in _SUBS:
                    pltpu.make_async_copy(
                        table_hbm.at[idx_v.at[s].at[pl.ds(off, ln)]],
                        rows_v.at[pl.ds(s * SEQ_L + off, ln)],
                        gsem,
                    ).wait()

        def process(rows_v):
            def add_body(r, c2):
                for s in range(SEQ_PER_CHUNK):
                    row = s * SEQ_L + r
                    for j in range(VECS_PER_ROW):
                        sl = pl.ds(j * LANES, LANES)
                        rows_v[row, sl] = rows_v[row, sl] + pe_v[r, sl]
                return c2

            lax.fori_loop(0, SEQ_L, add_body, 0)

        def fire_out(g, rows_v, osem):
            pltpu.async_copy(
                rows_v, out_hbm.at[pl.ds(base + g * CHUNK, CHUNK)], osem)

        def wait_out(g, rows_v, osem):
            pltpu.make_async_copy(
                rows_v, out_hbm.at[pl.ds(base + g * CHUNK, CHUNK)],
                osem).wait()

        last = NCHUNK - 1

        fire(0, idx_a, rows_a, gsem_a)
        fire(1, idx_b, rows_b, gsem_b)
        wait_g(idx_a, rows_a, gsem_a)
        process(rows_a)
        fire_out(0, rows_a, osem_a)
        wait_g(idx_b, rows_b, gsem_b)
        process(rows_b)
        fire_out(1, rows_b, osem_b)
        wait_out(0, rows_a, osem_a)
        fire(2, idx_a, rows_a, gsem_a)
        wait_out(1, rows_b, osem_b)
        fire(3, idx_b, rows_b, gsem_b)

        def pair_body(p, carry):
            g0 = 2 * p
            wait_g(idx_a, rows_a, gsem_a)
            process(rows_a)
            fire_out(g0, rows_a, osem_a)
            wait_g(idx_b, rows_b, gsem_b)
            process(rows_b)
            fire_out(g0 + 1, rows_b, osem_b)
            wait_out(g0, rows_a, osem_a)
            fire(jnp.minimum(g0 + 2, last), idx_a, rows_a, gsem_a)
            wait_out(g0 + 1, rows_b, osem_b)
            fire(jnp.minimum(g0 + 3, last), idx_b, rows_b, gsem_b)
            return carry

        lax.fori_loop(1, NCHUNK // 2, pair_body, 0)

        wait_g(idx_a, rows_a, gsem_a)
        wait_g(idx_b, rows_b, gsem_b)

    return body(table, idx_flat, pe)


def kernel(inputs, table):
    pe = jnp.asarray(_PE_CONST)
    out = _sc_embed(table, inputs, pe)
    return out.reshape(BATCH_N, SEQ_L, D_DIM)
